# speedup vs baseline: 1.1150x; 1.1150x over previous
import jax
import jax.numpy as jnp
from jax import lax
from jax.experimental import pallas as pl
from jax.experimental.pallas import tpu as pltpu
from jax.experimental.pallas import tpu_sc as plsc

VOCAB = 100000
D = 1024
B = 4 * 4096
NC = 2
NS = 16
NW = NC * NS
PER_W = B // NW
C = 16
NCHUNK = PER_W // C
LANES = 16
SCALE = 32.0


def _sc_body(idx_hbm, table_hbm, out_hbm, idx_v, in_buf, out_buf,
             gsem0, gsem1, osem0, osem1):
    gsems = (gsem0, gsem1)
    osems = (osem0, osem1)
    wid = lax.axis_index("s") * NC + lax.axis_index("c")
    row0 = wid * PER_W

    pltpu.sync_copy(idx_hbm.at[wid], idx_v)

    def start_gather(g, s):
        pltpu.async_copy(table_hbm.at[idx_v.at[g]], in_buf.at[s], gsems[s])

    def wait_gather(g, s):
        pltpu.make_async_copy(
            table_hbm.at[idx_v.at[g]], in_buf.at[s], gsems[s]).wait()

    def start_out(g, s):
        pltpu.async_copy(
            out_buf.at[s], out_hbm.at[pl.ds(row0 + g * C, C)], osems[s])

    def wait_out(g, s):
        pltpu.make_async_copy(
            out_buf.at[s], out_hbm.at[pl.ds(row0 + g * C, C)], osems[s]).wait()

    def scale(s):
        @pl.loop(0, C)
        def _(r):
            for c in range(D // LANES):
                sl = pl.ds(c * LANES, LANES)
                out_buf[s, r, sl] = in_buf[s, r, sl] * SCALE

    start_gather(0, 0)
    start_gather(1, 1)

    for s in range(2):
        wait_gather(s, s)
        scale(s)
        start_gather(s + 2, s)
        start_out(s, s)

    @pl.loop(1, NCHUNK // 2 - 1)
    def _(p):
        for s in range(2):
            g = p * 2 + s
            wait_gather(g, s)
            wait_out(g - 2, s)
            scale(s)
            start_gather(g + 2, s)
            start_out(g, s)

    for s in range(2):
        g = NCHUNK - 2 + s
        wait_gather(g, s)
        wait_out(g - 2, s)
        scale(s)
        start_out(g, s)
    for s in range(2):
        wait_out(NCHUNK - 2 + s, s)


def kernel(x, table):
    idx = x.reshape(NW, NCHUNK, C).astype(jnp.int32)
    call = pl.kernel(
        _sc_body,
        out_type=jax.ShapeDtypeStruct((B, D), jnp.float32),
        mesh=plsc.VectorSubcoreMesh(
            core_axis_name="c", subcore_axis_name="s",
            num_cores=NC, num_subcores=NS),
        scratch_types=[
            pltpu.VMEM((NCHUNK, C), jnp.int32),
            pltpu.VMEM((2, C, D), jnp.float32),
            pltpu.VMEM((2, C, D), jnp.float32),
            pltpu.SemaphoreType.DMA,
            pltpu.SemaphoreType.DMA,
            pltpu.SemaphoreType.DMA,
            pltpu.SemaphoreType.DMA,
        ],
    )
    out = call(idx, table)
    return out.reshape(x.shape + (D,))
